# baseline (device time: 31806 ns/iter reference)
import jax
import jax.numpy as jnp
from jax import lax
from jax.experimental import pallas as pl
from jax.experimental.pallas import tpu as pltpu

B, SQ, H, D = 4, 32, 8, 128
SKV_SHARD = 4096
N_SPLIT = 4
CHUNK = SKV_SHARD // N_SPLIT
SCALE = D ** -0.5

GROUPS = ((0, (0, 1, 2, 3)), (528, (4, 5, 6)), (928, (7,)))
GROUP_ROWS = (528, 400, 144)
TOT = 1072


def kernel(Q, K, V):
    def body(q_ref, k_ref, v_ref, out_ref,
             kbuf, vbuf, sendb, recv, sem_k, sem_v, send_s, recv_s):
        x = lax.axis_index("x")
        y = lax.axis_index("y")
        z = lax.axis_index("z")
        start = (2 * y + z) * CHUNK

        copies = [[] for _ in range(H)]
        for h in range(H):
            for b in range(B):
                copies[h].append(pltpu.make_async_copy(
                    k_ref.at[b, pl.ds(start, CHUNK), h, :], kbuf.at[b, h],
                    sem_k.at[h]))
                copies[h].append(pltpu.make_async_copy(
                    v_ref.at[b, pl.ds(start, CHUNK), h, :], vbuf.at[b, h],
                    sem_v.at[h]))
        for cs in copies:
            for c in cs:
                c.start()

        barrier = pltpu.get_barrier_semaphore()
        for nbr in ((1 - x, y, z), (x, 1 - y, z), (x, y, 1 - z)):
            pl.semaphore_signal(barrier, inc=1, device_id=nbr,
                                device_id_type=pl.DeviceIdType.MESH)
        pl.semaphore_wait(barrier, 3)

        ones_row = jnp.ones((1, CHUNK), jnp.float32)

        def compute_head(h):
            for c in copies[h]:
                c.wait()
            base, hs = next(g for g in GROUPS if h in g[1])
            npairs = B * len(hs)
            for b in range(B):
                loc = hs.index(h) * B + b
                qb = q_ref[b, :, h, :] * SCALE
                s = lax.dot_general(
                    qb, kbuf[b, h], (((1,), (1,)), ((), ())),
                    preferred_element_type=jnp.float32)
                p = jnp.exp(s)
                out_ref[pl.ds(base + loc * SQ, SQ), :] = lax.dot_general(
                    p, vbuf[b, h], (((1,), (0,)), ((), ())),
                    preferred_element_type=jnp.float32)
                den_row = lax.dot_general(
                    ones_row, p, (((1,), (1,)), ((), ())),
                    preferred_element_type=jnp.float32)
                out_ref[pl.ds(base + npairs * SQ + loc, 1), :] = jnp.pad(
                    den_row, ((0, 0), (0, D - SQ)))

        for g, (base, hs) in enumerate(GROUPS):
            used = B * len(hs) * (SQ + 1)
            pad = GROUP_ROWS[g] - used
            if pad:
                out_ref[pl.ds(base + used, pad), :] = jnp.zeros(
                    (pad, D), jnp.float32)

        nbrs = ((x, y, 1 - z), (x, 1 - y, z), (1 - x, y, z))
        live = {}

        def chain_start(g, p):
            base = GROUPS[g][0]
            gn = GROUP_ROWS[g]
            sl = pl.ds(base, gn)
            sendb[sl, :] = out_ref[sl, :].astype(jnp.bfloat16)
            a = (2 * g + p) % 3
            r = pltpu.make_async_remote_copy(
                src_ref=sendb.at[sl],
                dst_ref=recv.at[p, g, pl.ds(0, gn)],
                send_sem=send_s.at[p, g], recv_sem=recv_s.at[p, g],
                device_id=nbrs[a], device_id_type=pl.DeviceIdType.MESH)
            r.start()
            live[g] = (p, r)

        def chain_step(g):
            p, r = live[g]
            r.wait()
            base = GROUPS[g][0]
            gn = GROUP_ROWS[g]
            sl = pl.ds(base, gn)
            out_ref[sl, :] = out_ref[sl, :] + recv[
                p, g, pl.ds(0, gn)].astype(jnp.float32)
            if p < 2:
                chain_start(g, p + 1)

        compute_head(0)
        compute_head(1)
        compute_head(2)
        compute_head(3)
        chain_start(0, 0)
        compute_head(4)
        compute_head(5)
        chain_step(0)
        compute_head(6)
        chain_start(1, 0)
        compute_head(7)
        chain_start(2, 0)
        chain_step(0)
        chain_step(1)
        chain_step(2)
        chain_step(0)
        chain_step(1)
        chain_step(2)
        chain_step(1)
        chain_step(2)

    acc = pl.pallas_call(
        body,
        out_shape=jax.ShapeDtypeStruct((TOT, D), jnp.float32),
        in_specs=[
            pl.BlockSpec(memory_space=pltpu.VMEM),
            pl.BlockSpec(memory_space=pl.ANY),
            pl.BlockSpec(memory_space=pl.ANY),
        ],
        out_specs=pl.BlockSpec(memory_space=pltpu.VMEM),
        scratch_shapes=[
            pltpu.VMEM((B, H, CHUNK, D), jnp.float32),
            pltpu.VMEM((B, H, CHUNK, D), jnp.float32),
            pltpu.VMEM((TOT, D), jnp.bfloat16),
            pltpu.VMEM((3, 3, 528, D), jnp.bfloat16),
            pltpu.SemaphoreType.DMA((H,)),
            pltpu.SemaphoreType.DMA((H,)),
            pltpu.SemaphoreType.DMA((3, 3)),
            pltpu.SemaphoreType.DMA((3, 3)),
        ],
        compiler_params=pltpu.CompilerParams(
            collective_id=0,
            vmem_limit_bytes=100 * 1024 * 1024,
        ),
    )(Q, K, V)

    parts_num, parts_den = [], []
    for base, hs in GROUPS:
        n = B * len(hs)
        parts_num.append(acc[base:base + n * SQ].reshape(len(hs), B, SQ, D))
        parts_den.append(
            acc[base + n * SQ:base + n * SQ + n, :SQ].reshape(len(hs), B, SQ))
    num = jnp.concatenate(parts_num, axis=0)
    den = jnp.concatenate(parts_den, axis=0)
    out = num / den[..., None]
    return out.transpose(1, 2, 0, 3)


# device time: 27575 ns/iter; 1.1534x vs baseline; 1.1534x over previous
import jax
import jax.numpy as jnp
from jax import lax
from jax.experimental import pallas as pl
from jax.experimental.pallas import tpu as pltpu

B, SQ, H, D = 4, 32, 8, 128
SKV_SHARD = 4096
N_SPLIT = 4
CHUNK = SKV_SHARD // N_SPLIT
SCALE = D ** -0.5

GROUPS = ((0, (0, 1, 2, 3)), (528, (4, 5, 6)), (928, (7,)))
GROUP_ROWS = (528, 400, 144)
TOT = 1072


def kernel(Q, K, V):
    def body(q_ref, k_ref, v_ref, out_ref,
             accb, kbuf, vbuf, sendb, recv, sem_k, sem_v, send_s, recv_s):
        x = lax.axis_index("x")
        y = lax.axis_index("y")
        z = lax.axis_index("z")
        start = (2 * y + z) * CHUNK

        copies = [[] for _ in range(H)]
        for h in range(H):
            for b in range(B):
                copies[h].append(pltpu.make_async_copy(
                    k_ref.at[b, pl.ds(start, CHUNK), h, :], kbuf.at[b, h],
                    sem_k.at[h]))
                copies[h].append(pltpu.make_async_copy(
                    v_ref.at[b, pl.ds(start, CHUNK), h, :], vbuf.at[b, h],
                    sem_v.at[h]))
        for cs in copies:
            for c in cs:
                c.start()

        barrier = pltpu.get_barrier_semaphore()
        for nbr in ((1 - x, y, z), (x, 1 - y, z), (x, y, 1 - z)):
            pl.semaphore_signal(barrier, inc=1, device_id=nbr,
                                device_id_type=pl.DeviceIdType.MESH)
        pl.semaphore_wait(barrier, 3)

        ones_row = jnp.ones((1, CHUNK), jnp.float32)

        def compute_head(h):
            for c in copies[h]:
                c.wait()
            base, hs = next(g for g in GROUPS if h in g[1])
            npairs = B * len(hs)
            for b in range(B):
                loc = hs.index(h) * B + b
                qb = q_ref[b, :, h, :] * SCALE
                s = lax.dot_general(
                    qb, kbuf[b, h], (((1,), (1,)), ((), ())),
                    preferred_element_type=jnp.float32)
                p = jnp.exp(s)
                accb[pl.ds(base + loc * SQ, SQ), :] = lax.dot_general(
                    p, vbuf[b, h], (((1,), (0,)), ((), ())),
                    preferred_element_type=jnp.float32)
                den_row = lax.dot_general(
                    ones_row, p, (((1,), (1,)), ((), ())),
                    preferred_element_type=jnp.float32)
                accb[pl.ds(base + npairs * SQ + loc, 1), :] = jnp.pad(
                    den_row, ((0, 0), (0, D - SQ)))

        for g, (base, hs) in enumerate(GROUPS):
            used = B * len(hs) * (SQ + 1)
            pad = GROUP_ROWS[g] - used
            if pad:
                accb[pl.ds(base + used, pad), :] = jnp.zeros(
                    (pad, D), jnp.float32)

        nbrs = ((x, y, 1 - z), (x, 1 - y, z), (1 - x, y, z))
        live = {}

        def chain_start(g, p):
            base = GROUPS[g][0]
            gn = GROUP_ROWS[g]
            sl = pl.ds(base, gn)
            sendb[sl, :] = accb[sl, :].astype(jnp.bfloat16)
            a = (2 * g + p) % 3
            r = pltpu.make_async_remote_copy(
                src_ref=sendb.at[sl],
                dst_ref=recv.at[p, g, pl.ds(0, gn)],
                send_sem=send_s.at[p, g], recv_sem=recv_s.at[p, g],
                device_id=nbrs[a], device_id_type=pl.DeviceIdType.MESH)
            r.start()
            live[g] = (p, r)

        def chain_step(g):
            p, r = live[g]
            r.wait()
            base = GROUPS[g][0]
            gn = GROUP_ROWS[g]
            sl = pl.ds(base, gn)
            accb[sl, :] = accb[sl, :] + recv[
                p, g, pl.ds(0, gn)].astype(jnp.float32)
            if p < 2:
                chain_start(g, p + 1)

        compute_head(0)
        compute_head(1)
        compute_head(2)
        compute_head(3)
        chain_start(0, 0)
        compute_head(4)
        compute_head(5)
        chain_step(0)
        compute_head(6)
        chain_start(1, 0)
        compute_head(7)
        chain_start(2, 0)
        chain_step(0)
        chain_step(1)
        chain_step(2)
        chain_step(0)
        chain_step(1)
        chain_step(2)
        chain_step(1)
        chain_step(2)

        eye = jnp.eye(SQ, dtype=jnp.float32)
        for base, hs in GROUPS:
            npairs = B * len(hs)
            for h in hs:
                for b in range(B):
                    loc = hs.index(h) * B + b
                    num = accb[pl.ds(base + loc * SQ, SQ), :]
                    den_row = accb[
                        pl.ds(base + npairs * SQ + loc, 1), :SQ]
                    dmat = eye * (1.0 / den_row)
                    out_ref[b, :, h, :] = lax.dot_general(
                        dmat, num, (((1,), (0,)), ((), ())),
                        preferred_element_type=jnp.float32)

    return pl.pallas_call(
        body,
        out_shape=jax.ShapeDtypeStruct((B, SQ, H, D), jnp.float32),
        in_specs=[
            pl.BlockSpec(memory_space=pltpu.VMEM),
            pl.BlockSpec(memory_space=pl.ANY),
            pl.BlockSpec(memory_space=pl.ANY),
        ],
        out_specs=pl.BlockSpec(memory_space=pltpu.VMEM),
        scratch_shapes=[
            pltpu.VMEM((TOT, D), jnp.float32),
            pltpu.VMEM((B, H, CHUNK, D), jnp.float32),
            pltpu.VMEM((B, H, CHUNK, D), jnp.float32),
            pltpu.VMEM((TOT, D), jnp.bfloat16),
            pltpu.VMEM((3, 3, 528, D), jnp.bfloat16),
            pltpu.SemaphoreType.DMA((H,)),
            pltpu.SemaphoreType.DMA((H,)),
            pltpu.SemaphoreType.DMA((3, 3)),
            pltpu.SemaphoreType.DMA((3, 3)),
        ],
        compiler_params=pltpu.CompilerParams(
            collective_id=0,
            vmem_limit_bytes=100 * 1024 * 1024,
        ),
    )(Q, K, V)
